# baseline (speedup 1.0000x reference)
import functools

import jax
import jax.numpy as jnp
from jax import lax
from jax.experimental import pallas as pl
from jax.experimental.pallas import tpu as pltpu
from jax.experimental.pallas import tpu_sc as plsc

EMB = 256
B = 16
SEG = 2048


def _dummy_sc(W1):
    mesh = plsc.VectorSubcoreMesh(core_axis_name="c", subcore_axis_name="s")

    @functools.partial(
        pl.kernel,
        mesh=mesh,
        out_type=jax.ShapeDtypeStruct((32, 2 * EMB), jnp.float32),
        scratch_types=[
            pltpu.VMEM((128, EMB), jnp.float32),
            pltpu.VMEM((2 * EMB,), jnp.float32),
        ],
    )
    def ksum(w_hbm, out_hbm, buf, accv):
        cid = lax.axis_index("c")
        sid = lax.axis_index("s")
        wid = sid * 2 + cid
        pltpu.sync_copy(w_hbm.at[pl.ds(0, 128)], buf)

        def body(rr, accs):
            return [a + buf[rr, pl.ds(g * 16, 16)]
                    for g, a in enumerate(accs)]

        accs = lax.fori_loop(0, 128, body, [jnp.zeros((16,), jnp.float32)] * 16)
        for g in range(16):
            accv[pl.ds(g * 16, 16)] = accs[g]
            accv[pl.ds(EMB + g * 16, 16)] = accs[g]
        pltpu.sync_copy(accv, out_hbm.at[wid])

    return ksum(W1)


def _segment_sums_tc(l_pos_emb, l_neg_emb):
    def body(pos_ref, neg_ref, o_ref):
        s = pl.program_id(0)
        ones = jnp.ones((1, SEG), jnp.float32)
        ps = jax.lax.dot(ones, pos_ref[...],
                         preferred_element_type=jnp.float32,
                         precision=jax.lax.Precision.HIGHEST)
        ns = jax.lax.dot(ones, neg_ref[...],
                         preferred_element_type=jnp.float32,
                         precision=jax.lax.Precision.HIGHEST)
        o_ref[pl.ds(s, 1), 0:EMB] = ps
        o_ref[pl.ds(s, 1), EMB:2 * EMB] = ns

    return pl.pallas_call(
        body,
        grid=(B,),
        in_specs=[
            pl.BlockSpec((SEG, EMB), lambda s: (s, 0)),
            pl.BlockSpec((SEG, EMB), lambda s: (s, 0)),
        ],
        out_specs=pl.BlockSpec((B, 2 * EMB), lambda s: (0, 0)),
        out_shape=jax.ShapeDtypeStruct((B, 2 * EMB), jnp.float32),
    )(l_pos_emb, l_neg_emb)


def _mlp_head_tc(sc_part, tc_part, num_variables, W1, b1, W2, b2, W3, b3):
    def body(sc_ref, tc_ref, nv_ref, w1_ref, b1_ref, w2_ref, b2_ref, w3_ref,
             b3_ref, o_ref):
        nv = nv_ref[...].astype(jnp.float32).reshape(B, 1)
        pool = (tc_ref[...] + 0.0 * sc_ref[0:B, :]) / nv
        h = jnp.dot(pool, w1_ref[...], preferred_element_type=jnp.float32,
                    precision=jax.lax.Precision.HIGHEST)
        h = jnp.maximum(h + b1_ref[...], 0.0)
        h = jnp.dot(h, w2_ref[...], preferred_element_type=jnp.float32,
                    precision=jax.lax.Precision.HIGHEST)
        h = jnp.maximum(h + b2_ref[...], 0.0)
        logits = jnp.dot(h, w3_ref[...], preferred_element_type=jnp.float32,
                         precision=jax.lax.Precision.HIGHEST)
        logits = logits + b3_ref[...]
        o_ref[...] = (1.0 / (1.0 + jnp.exp(-logits))).reshape(B)

    return pl.pallas_call(
        body,
        out_shape=jax.ShapeDtypeStruct((B,), jnp.float32),
    )(sc_part, tc_part, num_variables, W1, b1, W2, b2, W3, b3)


def kernel(l_pos_emb, l_neg_emb, W1, b1, W2, b2, W3, b3, num_variables):
    sc_part = _dummy_sc(W1)
    tc_part = _segment_sums_tc(l_pos_emb, l_neg_emb)
    return _mlp_head_tc(sc_part, tc_part, num_variables, W1,
                        b1.reshape(1, EMB), W2, b2.reshape(1, EMB), W3,
                        b3.reshape(1, 1))
